# baseline (speedup 1.0000x reference)
import math

import jax
import jax.numpy as jnp
from jax import lax
from jax.experimental import pallas as pl
from jax.experimental.pallas import tpu as pltpu
from jax.experimental.pallas import tpu_sc as plsc

NP = 50000
NA = 50000
EW = 160000
EC = 160000
D = 128
H = 4
DH = 32
EDIM = 9

NSUB = 16
ECH = 80
EROWS = EW // ECH
SROWS = EROWS // NSUB
ROWS_A = 3200
ROWS_LAST = NP - 15 * ROWS_A
NPS = 50048

BLK = 1000
EBLK = 1280


def _ea_body(xt_ref, w_ref, b_ref, o_ref):
    o_ref[0] = lax.dot_general(
        xt_ref[...], w_ref[0], (((0,), (0,)), ((), ())),
        preferred_element_type=jnp.float32) + b_ref[0]


def _ea_proj(e_t, w_h, b_h):
    return pl.pallas_call(
        _ea_body,
        grid=(EW // EBLK, H),
        in_specs=[
            pl.BlockSpec((16, EBLK), lambda nb, h: (0, nb)),
            pl.BlockSpec((1, 16, DH), lambda nb, h: (h, 0, 0)),
            pl.BlockSpec((1, 1, DH), lambda nb, h: (h, 0, 0)),
        ],
        out_specs=pl.BlockSpec((1, EBLK, DH), lambda nb, h: (h, nb, 0)),
        out_shape=jax.ShapeDtypeStruct((H, EW, DH), jnp.float32),
    )(e_t, w_h, b_h)


def _paper_proj_body(x_ref, wqw_ref, bqw_ref, wqc_ref, bqc_ref,
                     wkv_ref, bkv_ref, qw_ref, qc_ref, kv_ref):
    x = x_ref[...]
    f32 = jnp.float32
    qw_ref[0] = jnp.dot(x, wqw_ref[0], preferred_element_type=f32) + bqw_ref[0]
    qc_ref[0] = jnp.dot(x, wqc_ref[0], preferred_element_type=f32) + bqc_ref[0]
    kv_ref[0] = jnp.dot(x, wkv_ref[0], preferred_element_type=f32) + bkv_ref[0]


def _paper_proj(x, wqw, bqw, wqc, bqc, wkv, bkv):
    grid = (NP // BLK, H)
    whspec = pl.BlockSpec((1, D, DH), lambda nb, h: (h, 0, 0))
    bhspec = pl.BlockSpec((1, 1, DH), lambda nb, h: (h, 0, 0))
    return pl.pallas_call(
        _paper_proj_body,
        grid=grid,
        in_specs=[
            pl.BlockSpec((BLK, D), lambda nb, h: (nb, 0)),
            whspec, bhspec, whspec, bhspec,
            pl.BlockSpec((1, D, 2 * DH), lambda nb, h: (h, 0, 0)),
            pl.BlockSpec((1, 1, 2 * DH), lambda nb, h: (h, 0, 0)),
        ],
        out_specs=[
            pl.BlockSpec((1, BLK, DH), lambda nb, h: (h, nb, 0)),
            pl.BlockSpec((1, BLK, DH), lambda nb, h: (h, nb, 0)),
            pl.BlockSpec((1, BLK, 2 * DH), lambda nb, h: (h, nb, 0)),
        ],
        out_shape=[
            jax.ShapeDtypeStruct((H, NP, DH), jnp.float32),
            jax.ShapeDtypeStruct((H, NP, DH), jnp.float32),
            jax.ShapeDtypeStruct((H, NP, 2 * DH), jnp.float32),
        ],
    )(x, wqw, bqw, wqc, bqc, wkv, bkv)


def _author_proj_body(x_ref, wkv_ref, bkv_ref, kv_ref):
    kv_ref[0] = jnp.dot(x_ref[...], wkv_ref[0],
                        preferred_element_type=jnp.float32) + bkv_ref[0]


def _author_proj(x, wkv, bkv):
    return pl.pallas_call(
        _author_proj_body,
        grid=(NA // BLK, H),
        in_specs=[
            pl.BlockSpec((BLK, D), lambda nb, h: (nb, 0)),
            pl.BlockSpec((1, D, 2 * DH), lambda nb, h: (h, 0, 0)),
            pl.BlockSpec((1, 1, 2 * DH), lambda nb, h: (h, 0, 0)),
        ],
        out_specs=pl.BlockSpec((1, BLK, 2 * DH), lambda nb, h: (h, nb, 0)),
        out_shape=jax.ShapeDtypeStruct((H, NA, 2 * DH), jnp.float32),
    )(x, wkv, bkv)


def _sc_edge_body(qw_hbm, qc_hbm, kvw_hbm, kvc_hbm, eaw_hbm, eac_hbm,
                  ixw_hbm, ixc_hbm, z32_hbm, z1_hbm,
                  raw_out, s_out,
                  idx0, idx1, q0, q1, kv0, kv1, ea0, ea1, contrib, wv,
                  gs0, gs1, is0, is1,
                  raw_acc, s_acc):
    core = lax.axis_index("c")
    sub = lax.axis_index("s")
    i32 = jnp.int32
    row0 = sub * SROWS

    def dual(do):
        @pl.when(sub < NSUB - 1)
        def _():
            do(pl.multiple_of(sub * ROWS_A, ROWS_A), ROWS_A, ROWS_A)

        @pl.when(sub == NSUB - 1)
        def _():
            do((NSUB - 1) * ROWS_A, ROWS_LAST, NPS - (NSUB - 1) * ROWS_A)

    for et in range(2):
        q_t = qw_hbm if et == 0 else qc_hbm
        kv_t = kvw_hbm if et == 0 else kvc_hbm
        ea_t = eaw_hbm if et == 0 else eac_hbm
        ix_t = ixw_hbm if et == 0 else ixc_hbm
        for hh in range(2):
            head = core * 2 + hh

            def zfill(off, n, ns):
                pltpu.sync_copy(z32_hbm.at[pl.ds(off, n)],
                                raw_acc.at[pl.ds(off, n)])
                pltpu.sync_copy(z1_hbm.at[pl.ds(off, ns)],
                                s_acc.at[pl.ds(off, ns)])
            dual(zfill)
            plsc.subcore_barrier()

            def idx_copy(c, ib, sem):
                cc = jnp.minimum(c, SROWS - 1)
                return pltpu.make_async_copy(ix_t.at[sub].at[cc], ib, sem)

            def gather_copies(ib, qb, kvb, eab, c, sem):
                return (
                    pltpu.make_async_copy(q_t.at[head].at[ib.at[0]], qb, sem),
                    pltpu.make_async_copy(kv_t.at[head].at[ib.at[1]], kvb, sem),
                    pltpu.make_async_copy(
                        ea_t.at[head].at[pl.ds((row0 + c) * ECH, ECH)],
                        eab, sem),
                )

            def issue_gathers(ib, qb, kvb, eab, c, sem):
                for dsc in gather_copies(ib, qb, kvb, eab, c, sem):
                    dsc.start()

            def wait_gathers(ib, qb, kvb, eab, c, sem):
                for dsc in gather_copies(ib, qb, kvb, eab, c, sem):
                    dsc.wait()

            def compute(qb, kvb, eab, ib, c):
                def group(g, carry2):
                    eids = lax.iota(i32, 16) + g * 16
                    acc = jnp.zeros((16,), jnp.float32)
                    for j in range(DH):
                        jj = jnp.full((16,), j, i32)
                        qv = plsc.load_gather(qb, [eids, jj])
                        kj = plsc.load_gather(kvb, [eids, jj])
                        ej = plsc.load_gather(eab, [eids, jj])
                        acc = acc + qv * (kj + ej)
                    w = jnp.exp(acc)
                    wv[pl.ds(g * 16, 16)] = w
                    for j in range(DH):
                        jj = jnp.full((16,), j, i32)
                        jv = jnp.full((16,), j + DH, i32)
                        vj = plsc.load_gather(kvb, [eids, jv])
                        ej = plsc.load_gather(eab, [eids, jj])
                        plsc.store_scatter(contrib, [eids, jj], w * (vj + ej))
                    return carry2

                lax.fori_loop(0, ECH // 16, group, 0)
                pltpu.sync_copy(contrib, raw_acc.at[ib.at[0]], add=True)
                pltpu.sync_copy(wv, s_acc.at[ib.at[0]], add=True)

            pltpu.sync_copy(ix_t.at[sub].at[0], idx0)
            issue_gathers(idx0, q0, kv0, ea0, 0, gs0)
            idx_copy(1, idx1, is1).start()

            def body2(t, carry):
                c0 = 2 * t
                c1 = c0 + 1
                idx_copy(c1, idx1, is1).wait()
                issue_gathers(idx1, q1, kv1, ea1, c1, gs1)
                wait_gathers(idx0, q0, kv0, ea0, c0, gs0)
                compute(q0, kv0, ea0, idx0, c0)
                idx_copy(c0 + 2, idx0, is0).start()
                idx_copy(c0 + 2, idx0, is0).wait()
                issue_gathers(idx0, q0, kv0, ea0, c0 + 2, gs0)
                wait_gathers(idx1, q1, kv1, ea1, c1, gs1)
                compute(q1, kv1, ea1, idx1, c1)
                idx_copy(c1 + 2, idx1, is1).start()
                return carry

            lax.fori_loop(0, (SROWS - 1) // 2, body2, 0)
            idx_copy(SROWS, idx1, is1).wait()
            clast = SROWS - 1
            wait_gathers(idx0, q0, kv0, ea0, clast, gs0)
            compute(q0, kv0, ea0, idx0, clast)

            plsc.subcore_barrier()
            oidx = et * H + head

            def wb(off, n, ns):
                pltpu.sync_copy(raw_acc.at[pl.ds(off, n)],
                                raw_out.at[oidx].at[pl.ds(off, n)])
                pltpu.sync_copy(s_acc.at[pl.ds(off, ns)],
                                s_out.at[oidx].at[pl.ds(off, ns)])
            dual(wb)
            plsc.subcore_barrier()


def _sc_edge(qw, qc, kvw, kvc, eaw, eac, ixw, ixc, z32, z1):
    mesh = plsc.VectorSubcoreMesh(core_axis_name="c", subcore_axis_name="s")
    fn = pl.kernel(
        _sc_edge_body,
        out_type=(
            jax.ShapeDtypeStruct((2 * H, NP, DH), jnp.float32),
            jax.ShapeDtypeStruct((2 * H, NPS), jnp.float32),
        ),
        mesh=mesh,
        compiler_params=pltpu.CompilerParams(
            needs_layout_passes=False, use_tc_tiling_on_sc=False),
        scratch_types=[
            pltpu.VMEM((2, ECH), jnp.int32),
            pltpu.VMEM((2, ECH), jnp.int32),
            pltpu.VMEM((ECH, DH), jnp.float32),
            pltpu.VMEM((ECH, DH), jnp.float32),
            pltpu.VMEM((ECH, 2 * DH), jnp.float32),
            pltpu.VMEM((ECH, 2 * DH), jnp.float32),
            pltpu.VMEM((ECH, DH), jnp.float32),
            pltpu.VMEM((ECH, DH), jnp.float32),
            pltpu.VMEM((ECH, DH), jnp.float32),
            pltpu.VMEM((ECH,), jnp.float32),
            pltpu.SemaphoreType.DMA,
            pltpu.SemaphoreType.DMA,
            pltpu.SemaphoreType.DMA,
            pltpu.SemaphoreType.DMA,
            pltpu.VMEM_SHARED((NP, DH), jnp.float32),
            pltpu.VMEM_SHARED((NPS,), jnp.float32),
        ],
    )
    return fn(qw, qc, kvw, kvc, eaw, eac, ixw, ixc, z32, z1)


def _post_body(raw_ref, s_ref, xp_ref, xa_ref, wa_ref, ba_ref, ombp_ref,
               gp_ref, bp_ref, abias_ref, omba_ref, ga_ref, bba_ref,
               hp_ref, ha_ref):
    f32 = jnp.float32
    o = None
    for h in range(H):
        rw = raw_ref[h]
        rc = raw_ref[H + h]
        sw = s_ref[:, h][:, None]
        sc_ = s_ref[:, H + h][:, None]
        agg = rw / (sw + 1e-16) + rc / (sc_ + 1e-16)
        g = jax.nn.gelu(agg)
        t = jnp.dot(g, wa_ref[h], preferred_element_type=f32)
        o = t if o is None else o + t
    res = o + ba_ref[...] + ombp_ref[...] * xp_ref[...]
    mu = jnp.mean(res, -1, keepdims=True)
    var = jnp.var(res, -1, keepdims=True)
    hp_ref[...] = gp_ref[...] * (res - mu) / jnp.sqrt(var + 1e-5) + bp_ref[...]

    ra = abias_ref[...] + omba_ref[...] * xa_ref[...]
    mua = jnp.mean(ra, -1, keepdims=True)
    vara = jnp.var(ra, -1, keepdims=True)
    ha_ref[...] = ga_ref[...] * (ra - mua) / jnp.sqrt(vara + 1e-5) + bba_ref[...]


def _post(raw, s, xp, xa, wa, ba, ombp, gp, bp, abias, omba, ga, bba):
    row = pl.BlockSpec((1, D), lambda nb: (0, 0))
    return pl.pallas_call(
        _post_body,
        grid=(NP // BLK,),
        in_specs=[
            pl.BlockSpec((2 * H, BLK, DH), lambda nb: (0, nb, 0)),
            pl.BlockSpec((BLK, 2 * H), lambda nb: (nb, 0)),
            pl.BlockSpec((BLK, D), lambda nb: (nb, 0)),
            pl.BlockSpec((BLK, D), lambda nb: (nb, 0)),
            pl.BlockSpec((H, DH, D), lambda nb: (0, 0, 0)),
            row, row, row, row, row, row, row, row,
        ],
        out_specs=[
            pl.BlockSpec((BLK, D), lambda nb: (nb, 0)),
            pl.BlockSpec((BLK, D), lambda nb: (nb, 0)),
        ],
        out_shape=[
            jax.ShapeDtypeStruct((NP, D), jnp.float32),
            jax.ShapeDtypeStruct((NA, D), jnp.float32),
        ],
    )(raw, s, xp, xa, wa, ba, ombp, gp, bp, abias, omba, ga, bba)


def kernel(x_paper, x_author, edge_index_writes, edge_index_cites,
           edge_t2v_writes, edge_t2v_cites, params):
    f32 = jnp.float32
    inv = 1.0 / math.sqrt(float(DH))

    def prep_ea(e, lin):
        e_t = jnp.pad(e.astype(f32), ((0, 0), (0, 16 - EDIM))).T
        w = jnp.pad(lin["w"].astype(f32), ((0, 16 - EDIM), (0, 0)))
        w_h = w.reshape(16, H, DH).transpose(1, 0, 2)
        b_h = lin["b"].astype(f32).reshape(H, 1, DH)
        return _ea_proj(e_t, w_h, b_h)

    eaw = prep_ea(edge_t2v_writes, params["edge_lin"]["writes"])
    eac = prep_ea(edge_t2v_cites, params["edge_lin"]["cites"])

    def prep_ix(ei):
        s_ = ei[0].astype(jnp.int32).reshape(NSUB, SROWS, ECH)
        d_ = ei[1].astype(jnp.int32).reshape(NSUB, SROWS, ECH)
        return jnp.stack([d_, s_], axis=2)

    ixw = prep_ix(edge_index_writes)
    ixc = prep_ix(edge_index_cites)

    z32 = jnp.zeros((NP, DH), f32)
    z1 = jnp.zeros((NPS,), f32)

    def per_head(w):
        return w.reshape(D, H, DH).transpose(1, 0, 2)

    def kv_weights(wk, bk, wv, bv, ar, mr):
        wk_eff = jnp.einsum("hkd,hdf->hkf", per_head(wk), ar)
        bk_eff = jnp.einsum("hd,hdf->hf", bk.reshape(H, DH), ar)
        wv_eff = jnp.einsum("hkd,hdf->hkf", per_head(wv), mr)
        bv_eff = jnp.einsum("hd,hdf->hf", bv.reshape(H, DH), mr)
        w = jnp.concatenate([wk_eff, wv_eff], axis=2)
        b = jnp.concatenate([bk_eff, bv_eff], axis=1).reshape(H, 1, 2 * DH)
        return w, b

    h_p = x_paper
    h_a = x_author
    for lp in params["layers"]:
        sc_w = (lp["p_rel"]["writes"] * inv)[:, None, None]
        sc_c = (lp["p_rel"]["cites"] * inv)[:, None, None]
        wq = per_head(lp["q"]["paper"]["w"])
        bq = lp["q"]["paper"]["b"].reshape(H, 1, DH)
        wkv_c, bkv_c = kv_weights(
            lp["k"]["paper"]["w"], lp["k"]["paper"]["b"],
            lp["v"]["paper"]["w"], lp["v"]["paper"]["b"],
            lp["a_rel"]["cites"], lp["m_rel"]["cites"])
        qw, qc, kvc = _paper_proj(h_p, wq * sc_w, bq * sc_w,
                                  wq * sc_c, bq * sc_c, wkv_c, bkv_c)

        wkv_w, bkv_w = kv_weights(
            lp["k"]["author"]["w"], lp["k"]["author"]["b"],
            lp["v"]["author"]["w"], lp["v"]["author"]["b"],
            lp["a_rel"]["writes"], lp["m_rel"]["writes"])
        kvw = _author_proj(h_a, wkv_w, bkv_w)

        raw, s = _sc_edge(qw, qc, kvw, kvc, eaw, eac, ixw, ixc, z32, z1)

        beta_p = jax.nn.sigmoid(lp["skip"]["paper"])
        beta_a = jax.nn.sigmoid(lp["skip"]["author"])
        wa = (lp["a"]["paper"]["w"].reshape(H, DH, D)) * beta_p
        ba = (lp["a"]["paper"]["b"] * beta_p).reshape(1, D)
        ombp = jnp.full((1, D), 1.0 - beta_p, f32)
        gp = params["norm"]["paper"]["g"].reshape(1, D)
        bp = params["norm"]["paper"]["b"].reshape(1, D)
        abias = (beta_a * lp["a"]["author"]["b"]).reshape(1, D)
        omba = jnp.full((1, D), 1.0 - beta_a, f32)
        ga = params["norm"]["author"]["g"].reshape(1, D)
        bba = params["norm"]["author"]["b"].reshape(1, D)

        h_p, h_a = _post(raw, s.transpose(1, 0)[:NP], h_p, h_a,
                         wa, ba, ombp, gp, bp, abias, omba, ga, bba)

    return (h_p, h_a)


# speedup vs baseline: 2.2428x; 2.2428x over previous
import math

import jax
import jax.numpy as jnp
from jax import lax
from jax.experimental import pallas as pl
from jax.experimental.pallas import tpu as pltpu
from jax.experimental.pallas import tpu_sc as plsc

NP = 50000
NA = 50000
EW = 160000
EC = 160000
D = 128
H = 4
DH = 32
EDIM = 9

NSUB = 16
ECH = 80
EROWS = EW // ECH
SROWS = EROWS // NSUB
ROWS_A = 3200
ROWS_LAST = NP - 15 * ROWS_A
NPS = 50048

BLK = 1000
EBLK = 1280


def _ea_body(xt_ref, w_ref, b_ref, o_ref):
    o_ref[...] = lax.dot_general(
        xt_ref[...], w_ref[...], (((0,), (0,)), ((), ())),
        preferred_element_type=jnp.float32) + b_ref[...]


def _ea_proj(e_t, w, b):
    return pl.pallas_call(
        _ea_body,
        grid=(EW // EBLK,),
        in_specs=[
            pl.BlockSpec((16, EBLK), lambda nb: (0, nb)),
            pl.BlockSpec((16, D), lambda nb: (0, 0)),
            pl.BlockSpec((1, D), lambda nb: (0, 0)),
        ],
        out_specs=pl.BlockSpec((EBLK, D), lambda nb: (nb, 0)),
        out_shape=jax.ShapeDtypeStruct((EW, D), jnp.float32),
    )(e_t, w, b)


def _paper_proj_body(x_ref, wqw_ref, bqw_ref, wqc_ref, bqc_ref,
                     wkv_ref, bkv_ref, qw_ref, qc_ref, kv_ref):
    x = x_ref[...]
    f32 = jnp.float32
    qw_ref[0] = jnp.dot(x, wqw_ref[0], preferred_element_type=f32) + bqw_ref[0]
    qc_ref[0] = jnp.dot(x, wqc_ref[0], preferred_element_type=f32) + bqc_ref[0]
    kv_ref[0] = jnp.dot(x, wkv_ref[0], preferred_element_type=f32) + bkv_ref[0]


def _paper_proj(x, wqw, bqw, wqc, bqc, wkv, bkv):
    grid = (NP // BLK, H)
    whspec = pl.BlockSpec((1, D, DH), lambda nb, h: (h, 0, 0))
    bhspec = pl.BlockSpec((1, 1, DH), lambda nb, h: (h, 0, 0))
    return pl.pallas_call(
        _paper_proj_body,
        grid=grid,
        in_specs=[
            pl.BlockSpec((BLK, D), lambda nb, h: (nb, 0)),
            whspec, bhspec, whspec, bhspec,
            pl.BlockSpec((1, D, 2 * DH), lambda nb, h: (h, 0, 0)),
            pl.BlockSpec((1, 1, 2 * DH), lambda nb, h: (h, 0, 0)),
        ],
        out_specs=[
            pl.BlockSpec((1, BLK, DH), lambda nb, h: (h, nb, 0)),
            pl.BlockSpec((1, BLK, DH), lambda nb, h: (h, nb, 0)),
            pl.BlockSpec((1, BLK, 2 * DH), lambda nb, h: (h, nb, 0)),
        ],
        out_shape=[
            jax.ShapeDtypeStruct((H, NP, DH), jnp.float32),
            jax.ShapeDtypeStruct((H, NP, DH), jnp.float32),
            jax.ShapeDtypeStruct((H, NP, 2 * DH), jnp.float32),
        ],
    )(x, wqw, bqw, wqc, bqc, wkv, bkv)


def _author_proj_body(x_ref, wkv_ref, bkv_ref, kv_ref):
    kv_ref[0] = jnp.dot(x_ref[...], wkv_ref[0],
                        preferred_element_type=jnp.float32) + bkv_ref[0]


def _author_proj(x, wkv, bkv):
    return pl.pallas_call(
        _author_proj_body,
        grid=(NA // BLK, H),
        in_specs=[
            pl.BlockSpec((BLK, D), lambda nb, h: (nb, 0)),
            pl.BlockSpec((1, D, 2 * DH), lambda nb, h: (h, 0, 0)),
            pl.BlockSpec((1, 1, 2 * DH), lambda nb, h: (h, 0, 0)),
        ],
        out_specs=pl.BlockSpec((1, BLK, 2 * DH), lambda nb, h: (h, nb, 0)),
        out_shape=jax.ShapeDtypeStruct((H, NA, 2 * DH), jnp.float32),
    )(x, wkv, bkv)


def _sc_edge_body(qw_hbm, qc_hbm, kvw_hbm, kvc_hbm, eaw_hbm, eac_hbm,
                  ixw_hbm, ixc_hbm, z32_hbm, z1_hbm,
                  raw_out, s_out,
                  idx0, idx1, q0, q1, kv0, kv1, ea0, ea1, contrib, wv,
                  gs0, gs1, is0, is1,
                  raw_acc, s_acc):
    core = lax.axis_index("c")
    sub = lax.axis_index("s")
    i32 = jnp.int32
    row0 = sub * SROWS

    def dual(do):
        @pl.when(sub < NSUB - 1)
        def _():
            do(pl.multiple_of(sub * ROWS_A, ROWS_A), ROWS_A, ROWS_A)

        @pl.when(sub == NSUB - 1)
        def _():
            do((NSUB - 1) * ROWS_A, ROWS_LAST, NPS - (NSUB - 1) * ROWS_A)

    for et in range(2):
        q_t = qw_hbm if et == 0 else qc_hbm
        kv_t = kvw_hbm if et == 0 else kvc_hbm
        ea_t = eaw_hbm if et == 0 else eac_hbm
        ix_t = ixw_hbm if et == 0 else ixc_hbm
        for hh in range(2):
            head = core * 2 + hh

            def zfill(off, n, ns):
                pltpu.sync_copy(z32_hbm.at[pl.ds(off, n)],
                                raw_acc.at[pl.ds(off, n)])
                pltpu.sync_copy(z1_hbm.at[pl.ds(off, ns)],
                                s_acc.at[pl.ds(off, ns)])
            dual(zfill)
            plsc.subcore_barrier()

            def idx_copy(c, ib, sem):
                cc = jnp.minimum(c, SROWS - 1)
                return pltpu.make_async_copy(ix_t.at[sub].at[cc], ib, sem)

            def gather_copies(ib, qb, kvb, eab, c, sem):
                return (
                    pltpu.make_async_copy(q_t.at[head].at[ib.at[0]], qb, sem),
                    pltpu.make_async_copy(kv_t.at[head].at[ib.at[1]], kvb, sem),
                    pltpu.make_async_copy(
                        ea_t.at[head].at[pl.ds((row0 + c) * ECH, ECH)],
                        eab, sem),
                )

            def issue_gathers(ib, qb, kvb, eab, c, sem):
                for dsc in gather_copies(ib, qb, kvb, eab, c, sem):
                    dsc.start()

            def wait_gathers(ib, qb, kvb, eab, c, sem):
                for dsc in gather_copies(ib, qb, kvb, eab, c, sem):
                    dsc.wait()

            def compute(qb, kvb, eab, ib, c):
                lane = lax.iota(i32, 16)

                def group(g, carry2):
                    asm = jnp.zeros((16,), jnp.float32)
                    for i in range(16):
                        e = g * 16 + i
                        qv0 = qb[e, pl.ds(0, 16)]
                        qv1 = qb[e, pl.ds(16, 16)]
                        k0 = kvb[e, pl.ds(0, 16)]
                        k1 = kvb[e, pl.ds(16, 16)]
                        eav0 = eab[e, pl.ds(0, 16)]
                        eav1 = eab[e, pl.ds(16, 16)]
                        p = qv0 * (k0 + eav0) + qv1 * (k1 + eav1)
                        tot = jnp.sum(p)
                        w = jnp.exp(jnp.full((16,), tot, jnp.float32))
                        asm = jnp.where(lane == i, w, asm)
                        v0 = kvb[e, pl.ds(32, 16)]
                        v1 = kvb[e, pl.ds(48, 16)]
                        contrib[e, pl.ds(0, 16)] = w * (v0 + eav0)
                        contrib[e, pl.ds(16, 16)] = w * (v1 + eav1)
                    wv[pl.ds(g * 16, 16)] = asm
                    return carry2

                lax.fori_loop(0, ECH // 16, group, 0)
                pltpu.sync_copy(contrib, raw_acc.at[ib.at[0]], add=True)
                pltpu.sync_copy(wv, s_acc.at[ib.at[0]], add=True)

            pltpu.sync_copy(ix_t.at[sub].at[0], idx0)
            issue_gathers(idx0, q0, kv0, ea0, 0, gs0)
            idx_copy(1, idx1, is1).start()

            def body2(t, carry):
                c0 = 2 * t
                c1 = c0 + 1
                idx_copy(c1, idx1, is1).wait()
                issue_gathers(idx1, q1, kv1, ea1, c1, gs1)
                wait_gathers(idx0, q0, kv0, ea0, c0, gs0)
                compute(q0, kv0, ea0, idx0, c0)
                idx_copy(c0 + 2, idx0, is0).start()
                idx_copy(c0 + 2, idx0, is0).wait()
                issue_gathers(idx0, q0, kv0, ea0, c0 + 2, gs0)
                wait_gathers(idx1, q1, kv1, ea1, c1, gs1)
                compute(q1, kv1, ea1, idx1, c1)
                idx_copy(c1 + 2, idx1, is1).start()
                return carry

            lax.fori_loop(0, (SROWS - 1) // 2, body2, 0)
            idx_copy(SROWS, idx1, is1).wait()
            clast = SROWS - 1
            wait_gathers(idx0, q0, kv0, ea0, clast, gs0)
            compute(q0, kv0, ea0, idx0, clast)

            plsc.subcore_barrier()
            oidx = et * H + head

            def wb(off, n, ns):
                pltpu.sync_copy(raw_acc.at[pl.ds(off, n)],
                                raw_out.at[oidx].at[pl.ds(off, n)])
                pltpu.sync_copy(s_acc.at[pl.ds(off, ns)],
                                s_out.at[oidx].at[pl.ds(off, ns)])
            dual(wb)
            plsc.subcore_barrier()


def _sc_edge(qw, qc, kvw, kvc, eaw, eac, ixw, ixc, z32, z1):
    mesh = plsc.VectorSubcoreMesh(core_axis_name="c", subcore_axis_name="s")
    fn = pl.kernel(
        _sc_edge_body,
        out_type=(
            jax.ShapeDtypeStruct((2 * H, NP, DH), jnp.float32),
            jax.ShapeDtypeStruct((2 * H, NPS), jnp.float32),
        ),
        mesh=mesh,
        compiler_params=pltpu.CompilerParams(
            needs_layout_passes=False, use_tc_tiling_on_sc=False),
        scratch_types=[
            pltpu.VMEM((2, ECH), jnp.int32),
            pltpu.VMEM((2, ECH), jnp.int32),
            pltpu.VMEM((ECH, DH), jnp.float32),
            pltpu.VMEM((ECH, DH), jnp.float32),
            pltpu.VMEM((ECH, 2 * DH), jnp.float32),
            pltpu.VMEM((ECH, 2 * DH), jnp.float32),
            pltpu.VMEM((ECH, DH), jnp.float32),
            pltpu.VMEM((ECH, DH), jnp.float32),
            pltpu.VMEM((ECH, DH), jnp.float32),
            pltpu.VMEM((ECH,), jnp.float32),
            pltpu.SemaphoreType.DMA,
            pltpu.SemaphoreType.DMA,
            pltpu.SemaphoreType.DMA,
            pltpu.SemaphoreType.DMA,
            pltpu.VMEM_SHARED((NP, DH), jnp.float32),
            pltpu.VMEM_SHARED((NPS,), jnp.float32),
        ],
    )
    return fn(qw, qc, kvw, kvc, eaw, eac, ixw, ixc, z32, z1)


def _post_body(raw_ref, s_ref, xp_ref, xa_ref, wa_ref, ba_ref, ombp_ref,
               gp_ref, bp_ref, abias_ref, omba_ref, ga_ref, bba_ref,
               hp_ref, ha_ref):
    f32 = jnp.float32
    o = None
    for h in range(H):
        rw = raw_ref[h]
        rc = raw_ref[H + h]
        sw = s_ref[:, h][:, None]
        sc_ = s_ref[:, H + h][:, None]
        agg = rw / (sw + 1e-16) + rc / (sc_ + 1e-16)
        g = jax.nn.gelu(agg)
        t = jnp.dot(g, wa_ref[h], preferred_element_type=f32)
        o = t if o is None else o + t
    res = o + ba_ref[...] + ombp_ref[...] * xp_ref[...]
    mu = jnp.mean(res, -1, keepdims=True)
    var = jnp.var(res, -1, keepdims=True)
    hp_ref[...] = gp_ref[...] * (res - mu) / jnp.sqrt(var + 1e-5) + bp_ref[...]

    ra = abias_ref[...] + omba_ref[...] * xa_ref[...]
    mua = jnp.mean(ra, -1, keepdims=True)
    vara = jnp.var(ra, -1, keepdims=True)
    ha_ref[...] = ga_ref[...] * (ra - mua) / jnp.sqrt(vara + 1e-5) + bba_ref[...]


def _post(raw, s, xp, xa, wa, ba, ombp, gp, bp, abias, omba, ga, bba):
    row = pl.BlockSpec((1, D), lambda nb: (0, 0))
    return pl.pallas_call(
        _post_body,
        grid=(NP // BLK,),
        in_specs=[
            pl.BlockSpec((2 * H, BLK, DH), lambda nb: (0, nb, 0)),
            pl.BlockSpec((BLK, 2 * H), lambda nb: (nb, 0)),
            pl.BlockSpec((BLK, D), lambda nb: (nb, 0)),
            pl.BlockSpec((BLK, D), lambda nb: (nb, 0)),
            pl.BlockSpec((H, DH, D), lambda nb: (0, 0, 0)),
            row, row, row, row, row, row, row, row,
        ],
        out_specs=[
            pl.BlockSpec((BLK, D), lambda nb: (nb, 0)),
            pl.BlockSpec((BLK, D), lambda nb: (nb, 0)),
        ],
        out_shape=[
            jax.ShapeDtypeStruct((NP, D), jnp.float32),
            jax.ShapeDtypeStruct((NA, D), jnp.float32),
        ],
    )(raw, s, xp, xa, wa, ba, ombp, gp, bp, abias, omba, ga, bba)


def kernel(x_paper, x_author, edge_index_writes, edge_index_cites,
           edge_t2v_writes, edge_t2v_cites, params):
    f32 = jnp.float32
    inv = 1.0 / math.sqrt(float(DH))

    def prep_ea(e, lin):
        e_t = jnp.pad(e.astype(f32), ((0, 0), (0, 16 - EDIM))).T
        w = jnp.pad(lin["w"].astype(f32), ((0, 16 - EDIM), (0, 0)))
        ea = _ea_proj(e_t, w, lin["b"].reshape(1, D))
        return ea.reshape(EW, H, DH).transpose(1, 0, 2)

    eaw = prep_ea(edge_t2v_writes, params["edge_lin"]["writes"])
    eac = prep_ea(edge_t2v_cites, params["edge_lin"]["cites"])

    def prep_ix(ei):
        s_ = ei[0].astype(jnp.int32).reshape(NSUB, SROWS, ECH)
        d_ = ei[1].astype(jnp.int32).reshape(NSUB, SROWS, ECH)
        return jnp.stack([d_, s_], axis=2)

    ixw = prep_ix(edge_index_writes)
    ixc = prep_ix(edge_index_cites)

    z32 = jnp.zeros((NP, DH), f32)
    z1 = jnp.zeros((NPS,), f32)

    def per_head(w):
        return w.reshape(D, H, DH).transpose(1, 0, 2)

    def kv_weights(wk, bk, wv, bv, ar, mr):
        wk_eff = jnp.einsum("hkd,hdf->hkf", per_head(wk), ar)
        bk_eff = jnp.einsum("hd,hdf->hf", bk.reshape(H, DH), ar)
        wv_eff = jnp.einsum("hkd,hdf->hkf", per_head(wv), mr)
        bv_eff = jnp.einsum("hd,hdf->hf", bv.reshape(H, DH), mr)
        w = jnp.concatenate([wk_eff, wv_eff], axis=2)
        b = jnp.concatenate([bk_eff, bv_eff], axis=1).reshape(H, 1, 2 * DH)
        return w, b

    h_p = x_paper
    h_a = x_author
    for lp in params["layers"]:
        sc_w = (lp["p_rel"]["writes"] * inv)[:, None, None]
        sc_c = (lp["p_rel"]["cites"] * inv)[:, None, None]
        wq = per_head(lp["q"]["paper"]["w"])
        bq = lp["q"]["paper"]["b"].reshape(H, 1, DH)
        wkv_c, bkv_c = kv_weights(
            lp["k"]["paper"]["w"], lp["k"]["paper"]["b"],
            lp["v"]["paper"]["w"], lp["v"]["paper"]["b"],
            lp["a_rel"]["cites"], lp["m_rel"]["cites"])
        qw, qc, kvc = _paper_proj(h_p, wq * sc_w, bq * sc_w,
                                  wq * sc_c, bq * sc_c, wkv_c, bkv_c)

        wkv_w, bkv_w = kv_weights(
            lp["k"]["author"]["w"], lp["k"]["author"]["b"],
            lp["v"]["author"]["w"], lp["v"]["author"]["b"],
            lp["a_rel"]["writes"], lp["m_rel"]["writes"])
        kvw = _author_proj(h_a, wkv_w, bkv_w)

        raw, s = _sc_edge(qw, qc, kvw, kvc, eaw, eac, ixw, ixc, z32, z1)

        beta_p = jax.nn.sigmoid(lp["skip"]["paper"])
        beta_a = jax.nn.sigmoid(lp["skip"]["author"])
        wa = (lp["a"]["paper"]["w"].reshape(H, DH, D)) * beta_p
        ba = (lp["a"]["paper"]["b"] * beta_p).reshape(1, D)
        ombp = jnp.full((1, D), 1.0 - beta_p, f32)
        gp = params["norm"]["paper"]["g"].reshape(1, D)
        bp = params["norm"]["paper"]["b"].reshape(1, D)
        abias = (beta_a * lp["a"]["author"]["b"]).reshape(1, D)
        omba = jnp.full((1, D), 1.0 - beta_a, f32)
        ga = params["norm"]["author"]["g"].reshape(1, D)
        bba = params["norm"]["author"]["b"].reshape(1, D)

        h_p, h_a = _post(raw, s.transpose(1, 0)[:NP], h_p, h_a,
                         wa, ba, ombp, gp, bp, abias, omba, ga, bba)

    return (h_p, h_a)


# speedup vs baseline: 2.3058x; 1.0281x over previous
import math

import jax
import jax.numpy as jnp
from jax import lax
from jax.experimental import pallas as pl
from jax.experimental.pallas import tpu as pltpu
from jax.experimental.pallas import tpu_sc as plsc

NP = 50000
NA = 50000
EW = 160000
EC = 160000
D = 128
H = 4
DH = 32
EDIM = 9

NSUB = 16
ECH = 80
EROWS = EW // ECH
SROWS = EROWS // NSUB
ROWS_A = 3200
ROWS_LAST = NP - 15 * ROWS_A
NPS = 50048

BLK = 1000
EBLK = 1280


def _ea_body(xt_ref, w_ref, b_ref, o_ref):
    o_ref[...] = lax.dot_general(
        xt_ref[...], w_ref[...], (((0,), (0,)), ((), ())),
        preferred_element_type=jnp.float32) + b_ref[...]


def _ea_proj(e_t, w, b):
    return pl.pallas_call(
        _ea_body,
        grid=(EW // EBLK,),
        in_specs=[
            pl.BlockSpec((16, EBLK), lambda nb: (0, nb)),
            pl.BlockSpec((16, D), lambda nb: (0, 0)),
            pl.BlockSpec((1, D), lambda nb: (0, 0)),
        ],
        out_specs=pl.BlockSpec((EBLK, D), lambda nb: (nb, 0)),
        out_shape=jax.ShapeDtypeStruct((EW, D), jnp.float32),
    )(e_t, w, b)


def _paper_proj_body(x_ref, w_ref, b_ref, qw_ref, qc_ref, kv_ref):
    y = jnp.dot(x_ref[...], w_ref[...],
                preferred_element_type=jnp.float32) + b_ref[...]
    qw_ref[...] = y[:, :D]
    qc_ref[...] = y[:, D:2 * D]
    kv_ref[...] = y[:, 2 * D:]


def _paper_proj(x, w, b):
    return pl.pallas_call(
        _paper_proj_body,
        grid=(NP // BLK,),
        in_specs=[
            pl.BlockSpec((BLK, D), lambda nb: (nb, 0)),
            pl.BlockSpec((D, 4 * D), lambda nb: (0, 0)),
            pl.BlockSpec((1, 4 * D), lambda nb: (0, 0)),
        ],
        out_specs=[
            pl.BlockSpec((BLK, D), lambda nb: (nb, 0)),
            pl.BlockSpec((BLK, D), lambda nb: (nb, 0)),
            pl.BlockSpec((BLK, 2 * D), lambda nb: (nb, 0)),
        ],
        out_shape=[
            jax.ShapeDtypeStruct((NP, D), jnp.float32),
            jax.ShapeDtypeStruct((NP, D), jnp.float32),
            jax.ShapeDtypeStruct((NP, 2 * D), jnp.float32),
        ],
    )(x, w, b)


def _author_proj_body(x_ref, w_ref, b_ref, kv_ref):
    kv_ref[...] = jnp.dot(x_ref[...], w_ref[...],
                          preferred_element_type=jnp.float32) + b_ref[...]


def _author_proj(x, w, b):
    return pl.pallas_call(
        _author_proj_body,
        grid=(NA // BLK,),
        in_specs=[
            pl.BlockSpec((BLK, D), lambda nb: (nb, 0)),
            pl.BlockSpec((D, 2 * D), lambda nb: (0, 0)),
            pl.BlockSpec((1, 2 * D), lambda nb: (0, 0)),
        ],
        out_specs=pl.BlockSpec((BLK, 2 * D), lambda nb: (nb, 0)),
        out_shape=jax.ShapeDtypeStruct((NA, 2 * D), jnp.float32),
    )(x, w, b)


def _sc_edge_body(qw_hbm, qc_hbm, kvw_hbm, kvc_hbm, eaw_hbm, eac_hbm,
                  ixw_hbm, ixc_hbm, z32_hbm, z1_hbm,
                  raw_out, s_out,
                  idx0, idx1, q0, q1, kv0, kv1, ea0, ea1, contrib, wv,
                  gs0, gs1, is0, is1,
                  raw_acc, s_acc):
    core = lax.axis_index("c")
    sub = lax.axis_index("s")
    i32 = jnp.int32
    row0 = sub * SROWS

    def dual(do):
        @pl.when(sub < NSUB - 1)
        def _():
            do(pl.multiple_of(sub * ROWS_A, ROWS_A), ROWS_A, ROWS_A)

        @pl.when(sub == NSUB - 1)
        def _():
            do((NSUB - 1) * ROWS_A, ROWS_LAST, NPS - (NSUB - 1) * ROWS_A)

    for et in range(2):
        q_t = qw_hbm if et == 0 else qc_hbm
        kv_t = kvw_hbm if et == 0 else kvc_hbm
        ea_t = eaw_hbm if et == 0 else eac_hbm
        ix_t = ixw_hbm if et == 0 else ixc_hbm
        for hh in range(2):
            head = core * 2 + hh

            def zfill(off, n, ns):
                pltpu.sync_copy(z32_hbm.at[pl.ds(off, n)],
                                raw_acc.at[pl.ds(off, n)])
                pltpu.sync_copy(z1_hbm.at[pl.ds(off, ns)],
                                s_acc.at[pl.ds(off, ns)])
            dual(zfill)
            plsc.subcore_barrier()

            def idx_copy(c, ib, sem):
                cc = jnp.minimum(c, SROWS - 1)
                return pltpu.make_async_copy(ix_t.at[sub].at[cc], ib, sem)

            def gather_copies(ib, qb, kvb, eab, c, sem):
                return (
                    pltpu.make_async_copy(q_t.at[head].at[ib.at[0]], qb, sem),
                    pltpu.make_async_copy(kv_t.at[head].at[ib.at[1]], kvb, sem),
                    pltpu.make_async_copy(
                        ea_t.at[head].at[pl.ds((row0 + c) * ECH, ECH)],
                        eab, sem),
                )

            def issue_gathers(ib, qb, kvb, eab, c, sem):
                for dsc in gather_copies(ib, qb, kvb, eab, c, sem):
                    dsc.start()

            def wait_gathers(ib, qb, kvb, eab, c, sem):
                for dsc in gather_copies(ib, qb, kvb, eab, c, sem):
                    dsc.wait()

            def compute(qb, kvb, eab, ib, c):
                lane = lax.iota(i32, 16)

                def group(g, carry2):
                    asm = jnp.zeros((16,), jnp.float32)
                    for i in range(16):
                        e = g * 16 + i
                        qv0 = qb[e, pl.ds(0, 16)]
                        qv1 = qb[e, pl.ds(16, 16)]
                        k0 = kvb[e, pl.ds(0, 16)]
                        k1 = kvb[e, pl.ds(16, 16)]
                        eav0 = eab[e, pl.ds(0, 16)]
                        eav1 = eab[e, pl.ds(16, 16)]
                        p = qv0 * (k0 + eav0) + qv1 * (k1 + eav1)
                        tot = jnp.sum(p)
                        w = jnp.exp(jnp.full((16,), tot, jnp.float32))
                        asm = jnp.where(lane == i, w, asm)
                        v0 = kvb[e, pl.ds(32, 16)]
                        v1 = kvb[e, pl.ds(48, 16)]
                        contrib[e, pl.ds(0, 16)] = w * (v0 + eav0)
                        contrib[e, pl.ds(16, 16)] = w * (v1 + eav1)
                    wv[pl.ds(g * 16, 16)] = asm
                    return carry2

                lax.fori_loop(0, ECH // 16, group, 0)
                pltpu.sync_copy(contrib, raw_acc.at[ib.at[0]], add=True)
                pltpu.sync_copy(wv, s_acc.at[ib.at[0]], add=True)

            pltpu.sync_copy(ix_t.at[sub].at[0], idx0)
            issue_gathers(idx0, q0, kv0, ea0, 0, gs0)
            idx_copy(1, idx1, is1).start()

            def body2(t, carry):
                c0 = 2 * t
                c1 = c0 + 1
                idx_copy(c1, idx1, is1).wait()
                issue_gathers(idx1, q1, kv1, ea1, c1, gs1)
                wait_gathers(idx0, q0, kv0, ea0, c0, gs0)
                compute(q0, kv0, ea0, idx0, c0)
                idx_copy(c0 + 2, idx0, is0).start()
                idx_copy(c0 + 2, idx0, is0).wait()
                issue_gathers(idx0, q0, kv0, ea0, c0 + 2, gs0)
                wait_gathers(idx1, q1, kv1, ea1, c1, gs1)
                compute(q1, kv1, ea1, idx1, c1)
                idx_copy(c1 + 2, idx1, is1).start()
                return carry

            lax.fori_loop(0, (SROWS - 1) // 2, body2, 0)
            idx_copy(SROWS, idx1, is1).wait()
            clast = SROWS - 1
            wait_gathers(idx0, q0, kv0, ea0, clast, gs0)
            compute(q0, kv0, ea0, idx0, clast)

            plsc.subcore_barrier()
            oidx = et * H + head

            def wb(off, n, ns):
                pltpu.sync_copy(raw_acc.at[pl.ds(off, n)],
                                raw_out.at[oidx].at[pl.ds(off, n)])
                pltpu.sync_copy(s_acc.at[pl.ds(off, ns)],
                                s_out.at[oidx].at[pl.ds(off, ns)])
            dual(wb)
            plsc.subcore_barrier()


def _sc_edge(qw, qc, kvw, kvc, eaw, eac, ixw, ixc, z32, z1):
    mesh = plsc.VectorSubcoreMesh(core_axis_name="c", subcore_axis_name="s")
    fn = pl.kernel(
        _sc_edge_body,
        out_type=(
            jax.ShapeDtypeStruct((2 * H, NP, DH), jnp.float32),
            jax.ShapeDtypeStruct((2 * H, NPS), jnp.float32),
        ),
        mesh=mesh,
        compiler_params=pltpu.CompilerParams(
            needs_layout_passes=False, use_tc_tiling_on_sc=False),
        scratch_types=[
            pltpu.VMEM((2, ECH), jnp.int32),
            pltpu.VMEM((2, ECH), jnp.int32),
            pltpu.VMEM((ECH, DH), jnp.float32),
            pltpu.VMEM((ECH, DH), jnp.float32),
            pltpu.VMEM((ECH, 2 * DH), jnp.float32),
            pltpu.VMEM((ECH, 2 * DH), jnp.float32),
            pltpu.VMEM((ECH, DH), jnp.float32),
            pltpu.VMEM((ECH, DH), jnp.float32),
            pltpu.VMEM((ECH, DH), jnp.float32),
            pltpu.VMEM((ECH,), jnp.float32),
            pltpu.SemaphoreType.DMA,
            pltpu.SemaphoreType.DMA,
            pltpu.SemaphoreType.DMA,
            pltpu.SemaphoreType.DMA,
            pltpu.VMEM_SHARED((NP, DH), jnp.float32),
            pltpu.VMEM_SHARED((NPS,), jnp.float32),
        ],
    )
    return fn(qw, qc, kvw, kvc, eaw, eac, ixw, ixc, z32, z1)


def _post_body(raw_ref, s_ref, xp_ref, xa_ref, wa_ref, ba_ref, ombp_ref,
               gp_ref, bp_ref, abias_ref, omba_ref, ga_ref, bba_ref,
               hp_ref, ha_ref):
    f32 = jnp.float32
    o = None
    for h in range(H):
        rw = raw_ref[h]
        rc = raw_ref[H + h]
        sw = s_ref[:, h][:, None]
        sc_ = s_ref[:, H + h][:, None]
        agg = rw / (sw + 1e-16) + rc / (sc_ + 1e-16)
        g = jax.nn.gelu(agg)
        t = jnp.dot(g, wa_ref[h], preferred_element_type=f32)
        o = t if o is None else o + t
    res = o + ba_ref[...] + ombp_ref[...] * xp_ref[...]
    mu = jnp.mean(res, -1, keepdims=True)
    var = jnp.var(res, -1, keepdims=True)
    hp_ref[...] = gp_ref[...] * (res - mu) / jnp.sqrt(var + 1e-5) + bp_ref[...]

    ra = abias_ref[...] + omba_ref[...] * xa_ref[...]
    mua = jnp.mean(ra, -1, keepdims=True)
    vara = jnp.var(ra, -1, keepdims=True)
    ha_ref[...] = ga_ref[...] * (ra - mua) / jnp.sqrt(vara + 1e-5) + bba_ref[...]


def _post(raw, s, xp, xa, wa, ba, ombp, gp, bp, abias, omba, ga, bba):
    row = pl.BlockSpec((1, D), lambda nb: (0, 0))
    return pl.pallas_call(
        _post_body,
        grid=(NP // BLK,),
        in_specs=[
            pl.BlockSpec((2 * H, BLK, DH), lambda nb: (0, nb, 0)),
            pl.BlockSpec((BLK, 2 * H), lambda nb: (nb, 0)),
            pl.BlockSpec((BLK, D), lambda nb: (nb, 0)),
            pl.BlockSpec((BLK, D), lambda nb: (nb, 0)),
            pl.BlockSpec((H, DH, D), lambda nb: (0, 0, 0)),
            row, row, row, row, row, row, row, row,
        ],
        out_specs=[
            pl.BlockSpec((BLK, D), lambda nb: (nb, 0)),
            pl.BlockSpec((BLK, D), lambda nb: (nb, 0)),
        ],
        out_shape=[
            jax.ShapeDtypeStruct((NP, D), jnp.float32),
            jax.ShapeDtypeStruct((NA, D), jnp.float32),
        ],
    )(raw, s, xp, xa, wa, ba, ombp, gp, bp, abias, omba, ga, bba)


def kernel(x_paper, x_author, edge_index_writes, edge_index_cites,
           edge_t2v_writes, edge_t2v_cites, params):
    f32 = jnp.float32
    inv = 1.0 / math.sqrt(float(DH))

    def prep_ea(e, lin):
        e_t = jnp.pad(e.astype(f32), ((0, 0), (0, 16 - EDIM))).T
        w = jnp.pad(lin["w"].astype(f32), ((0, 16 - EDIM), (0, 0)))
        ea = _ea_proj(e_t, w, lin["b"].reshape(1, D))
        return ea.reshape(EW, H, DH).transpose(1, 0, 2)

    eaw = prep_ea(edge_t2v_writes, params["edge_lin"]["writes"])
    eac = prep_ea(edge_t2v_cites, params["edge_lin"]["cites"])

    def prep_ix(ei):
        s_ = ei[0].astype(jnp.int32).reshape(NSUB, SROWS, ECH)
        d_ = ei[1].astype(jnp.int32).reshape(NSUB, SROWS, ECH)
        return jnp.stack([d_, s_], axis=2)

    ixw = prep_ix(edge_index_writes)
    ixc = prep_ix(edge_index_cites)

    z32 = jnp.zeros((NP, DH), f32)
    z1 = jnp.zeros((NPS,), f32)

    def per_head(w):
        return w.reshape(D, H, DH).transpose(1, 0, 2)

    def kv_weights(wk, bk, wv, bv, ar, mr):
        wk_eff = jnp.einsum("hkd,hdf->hkf", per_head(wk), ar)
        bk_eff = jnp.einsum("hd,hdf->hf", bk.reshape(H, DH), ar)
        wv_eff = jnp.einsum("hkd,hdf->hkf", per_head(wv), mr)
        bv_eff = jnp.einsum("hd,hdf->hf", bv.reshape(H, DH), mr)
        w = jnp.concatenate([wk_eff, wv_eff], axis=2)
        w = w.transpose(1, 0, 2).reshape(D, 2 * D)
        b = jnp.concatenate([bk_eff, bv_eff], axis=1).reshape(1, 2 * D)
        return w, b

    def headify(t, width):
        return t.reshape(-1, H, width).transpose(1, 0, 2)

    h_p = x_paper
    h_a = x_author
    for lp in params["layers"]:
        sv_w = jnp.repeat(lp["p_rel"]["writes"] * inv, DH)
        sv_c = jnp.repeat(lp["p_rel"]["cites"] * inv, DH)
        wq = lp["q"]["paper"]["w"]
        bq = lp["q"]["paper"]["b"]
        wkv_c, bkv_c = kv_weights(
            lp["k"]["paper"]["w"], lp["k"]["paper"]["b"],
            lp["v"]["paper"]["w"], lp["v"]["paper"]["b"],
            lp["a_rel"]["cites"], lp["m_rel"]["cites"])
        w_paper = jnp.concatenate(
            [wq * sv_w[None, :], wq * sv_c[None, :], wkv_c], axis=1)
        b_paper = jnp.concatenate(
            [(bq * sv_w).reshape(1, D), (bq * sv_c).reshape(1, D), bkv_c],
            axis=1)
        qw0, qc0, kvc0 = _paper_proj(h_p, w_paper, b_paper)
        qw = headify(qw0, DH)
        qc = headify(qc0, DH)
        kvc = headify(kvc0, 2 * DH)

        wkv_w, bkv_w = kv_weights(
            lp["k"]["author"]["w"], lp["k"]["author"]["b"],
            lp["v"]["author"]["w"], lp["v"]["author"]["b"],
            lp["a_rel"]["writes"], lp["m_rel"]["writes"])
        kvw = headify(_author_proj(h_a, wkv_w, bkv_w), 2 * DH)

        raw, s = _sc_edge(qw, qc, kvw, kvc, eaw, eac, ixw, ixc, z32, z1)

        beta_p = jax.nn.sigmoid(lp["skip"]["paper"])
        beta_a = jax.nn.sigmoid(lp["skip"]["author"])
        wa = (lp["a"]["paper"]["w"].reshape(H, DH, D)) * beta_p
        ba = (lp["a"]["paper"]["b"] * beta_p).reshape(1, D)
        ombp = jnp.full((1, D), 1.0 - beta_p, f32)
        gp = params["norm"]["paper"]["g"].reshape(1, D)
        bp = params["norm"]["paper"]["b"].reshape(1, D)
        abias = (beta_a * lp["a"]["author"]["b"]).reshape(1, D)
        omba = jnp.full((1, D), 1.0 - beta_a, f32)
        ga = params["norm"]["author"]["g"].reshape(1, D)
        bba = params["norm"]["author"]["b"].reshape(1, D)

        h_p, h_a = _post(raw, s.transpose(1, 0)[:NP], h_p, h_a,
                         wa, ba, ombp, gp, bp, abias, omba, ga, bba)

    return (h_p, h_a)


# speedup vs baseline: 2.3599x; 1.0235x over previous
import math

import jax
import jax.numpy as jnp
from jax import lax
from jax.experimental import pallas as pl
from jax.experimental.pallas import tpu as pltpu
from jax.experimental.pallas import tpu_sc as plsc

NP = 50000
NA = 50000
EW = 160000
EC = 160000
D = 128
H = 4
DH = 32
EDIM = 9

NSUB = 16
ECH = 80
EROWS = EW // ECH
SROWS = EROWS // NSUB
ROWS_A = 3200
ROWS_LAST = NP - 15 * ROWS_A
NPS = 50048

BLK = 1000
EBLK = 1280


def _ea_body(xt_ref, w_ref, b_ref, o_ref):
    o_ref[...] = lax.dot_general(
        xt_ref[...], w_ref[...], (((0,), (0,)), ((), ())),
        preferred_element_type=jnp.float32) + b_ref[...]


def _ea_proj(e_t, w, b):
    return pl.pallas_call(
        _ea_body,
        grid=(EW // EBLK,),
        in_specs=[
            pl.BlockSpec((16, EBLK), lambda nb: (0, nb)),
            pl.BlockSpec((16, D), lambda nb: (0, 0)),
            pl.BlockSpec((1, D), lambda nb: (0, 0)),
        ],
        out_specs=pl.BlockSpec((EBLK, D), lambda nb: (nb, 0)),
        out_shape=jax.ShapeDtypeStruct((EW, D), jnp.float32),
    )(e_t, w, b)


def _paper_proj_body(x_ref, w_ref, b_ref, qw_ref, qc_ref, kv_ref):
    y = jnp.dot(x_ref[...], w_ref[...],
                preferred_element_type=jnp.float32) + b_ref[...]
    qw_ref[...] = y[:, :D]
    qc_ref[...] = y[:, D:2 * D]
    kv_ref[...] = y[:, 2 * D:]


def _paper_proj(x, w, b):
    return pl.pallas_call(
        _paper_proj_body,
        grid=(NP // BLK,),
        in_specs=[
            pl.BlockSpec((BLK, D), lambda nb: (nb, 0)),
            pl.BlockSpec((D, 4 * D), lambda nb: (0, 0)),
            pl.BlockSpec((1, 4 * D), lambda nb: (0, 0)),
        ],
        out_specs=[
            pl.BlockSpec((BLK, D), lambda nb: (nb, 0)),
            pl.BlockSpec((BLK, D), lambda nb: (nb, 0)),
            pl.BlockSpec((BLK, 2 * D), lambda nb: (nb, 0)),
        ],
        out_shape=[
            jax.ShapeDtypeStruct((NP, D), jnp.float32),
            jax.ShapeDtypeStruct((NP, D), jnp.float32),
            jax.ShapeDtypeStruct((NP, 2 * D), jnp.float32),
        ],
    )(x, w, b)


def _author_proj_body(x_ref, w_ref, b_ref, kv_ref):
    kv_ref[...] = jnp.dot(x_ref[...], w_ref[...],
                          preferred_element_type=jnp.float32) + b_ref[...]


def _author_proj(x, w, b):
    return pl.pallas_call(
        _author_proj_body,
        grid=(NA // BLK,),
        in_specs=[
            pl.BlockSpec((BLK, D), lambda nb: (nb, 0)),
            pl.BlockSpec((D, 2 * D), lambda nb: (0, 0)),
            pl.BlockSpec((1, 2 * D), lambda nb: (0, 0)),
        ],
        out_specs=pl.BlockSpec((BLK, 2 * D), lambda nb: (nb, 0)),
        out_shape=jax.ShapeDtypeStruct((NA, 2 * D), jnp.float32),
    )(x, w, b)


def _sc_edge_body(qw_hbm, qc_hbm, kvw_hbm, kvc_hbm, eaw_hbm, eac_hbm,
                  ixw_hbm, ixc_hbm, z32_hbm, z1_hbm,
                  raw_out, s_out,
                  idx0, idx1, q0, q1, kv0, kv1, ea0, ea1,
                  contrib0, contrib1, wv0, wv1, sidx0, sidx1,
                  gs0, gs1, is0, is1, ss0, ss1,
                  raw_acc, s_acc):
    core = lax.axis_index("c")
    sub = lax.axis_index("s")
    i32 = jnp.int32
    row0 = sub * SROWS

    def dual(do):
        @pl.when(sub < NSUB - 1)
        def _():
            do(pl.multiple_of(sub * ROWS_A, ROWS_A), ROWS_A, ROWS_A)

        @pl.when(sub == NSUB - 1)
        def _():
            do((NSUB - 1) * ROWS_A, ROWS_LAST, NPS - (NSUB - 1) * ROWS_A)

    for et in range(2):
        q_t = qw_hbm if et == 0 else qc_hbm
        kv_t = kvw_hbm if et == 0 else kvc_hbm
        ea_t = eaw_hbm if et == 0 else eac_hbm
        ix_t = ixw_hbm if et == 0 else ixc_hbm
        for hh in range(2):
            head = core * 2 + hh

            def zfill(off, n, ns):
                pltpu.sync_copy(z32_hbm.at[pl.ds(off, n)],
                                raw_acc.at[pl.ds(off, n)])
                pltpu.sync_copy(z1_hbm.at[pl.ds(off, ns)],
                                s_acc.at[pl.ds(off, ns)])
            dual(zfill)
            plsc.subcore_barrier()

            def idx_copy(c, ib, sem):
                cc = jnp.minimum(c, SROWS - 1)
                return pltpu.make_async_copy(ix_t.at[sub].at[cc], ib, sem)

            def gather_copies(ib, qb, kvb, eab, c, sem):
                return (
                    pltpu.make_async_copy(q_t.at[head].at[ib.at[0]], qb, sem),
                    pltpu.make_async_copy(kv_t.at[head].at[ib.at[1]], kvb, sem),
                    pltpu.make_async_copy(
                        ea_t.at[head].at[pl.ds((row0 + c) * ECH, ECH)],
                        eab, sem),
                )

            def issue_gathers(ib, qb, kvb, eab, c, sem):
                for dsc in gather_copies(ib, qb, kvb, eab, c, sem):
                    dsc.start()

            def wait_gathers(ib, qb, kvb, eab, c, sem):
                for dsc in gather_copies(ib, qb, kvb, eab, c, sem):
                    dsc.wait()

            def scatter_copies(cb, wb_, sx, sem):
                return (
                    pltpu.make_async_copy(cb, raw_acc.at[sx], sem),
                    pltpu.make_async_copy(wb_, s_acc.at[sx], sem),
                )

            def wait_scatters(cb, wb_, sx, sem):
                for dsc in scatter_copies(cb, wb_, sx, sem):
                    dsc.wait()

            def compute(qb, kvb, eab, ib, cb, wb_, sx, sem):
                lane = lax.iota(i32, 16)

                def group(g, carry2):
                    asm = jnp.zeros((16,), jnp.float32)
                    sidx_v = ib[0, pl.ds(g * 16, 16)]
                    sx[pl.ds(g * 16, 16)] = sidx_v
                    for i in range(16):
                        e = g * 16 + i
                        qv0 = qb[e, pl.ds(0, 16)]
                        qv1 = qb[e, pl.ds(16, 16)]
                        k0 = kvb[e, pl.ds(0, 16)]
                        k1 = kvb[e, pl.ds(16, 16)]
                        eav0 = eab[e, pl.ds(0, 16)]
                        eav1 = eab[e, pl.ds(16, 16)]
                        p = qv0 * (k0 + eav0) + qv1 * (k1 + eav1)
                        tot = jnp.sum(p)
                        w = jnp.exp(jnp.full((16,), tot, jnp.float32))
                        asm = jnp.where(lane == i, w, asm)
                        v0 = kvb[e, pl.ds(32, 16)]
                        v1 = kvb[e, pl.ds(48, 16)]
                        cb[e, pl.ds(0, 16)] = w * (v0 + eav0)
                        cb[e, pl.ds(16, 16)] = w * (v1 + eav1)
                    wb_[pl.ds(g * 16, 16)] = asm
                    return carry2

                lax.fori_loop(0, ECH // 16, group, 0)
                for dsc in scatter_copies(cb, wb_, sx, sem):
                    dsc.start()

            pltpu.sync_copy(ix_t.at[sub].at[0], idx0)
            issue_gathers(idx0, q0, kv0, ea0, 0, gs0)
            idx_copy(1, idx1, is1).start()
            idx_copy(1, idx1, is1).wait()
            issue_gathers(idx1, q1, kv1, ea1, 1, gs1)
            wait_gathers(idx0, q0, kv0, ea0, 0, gs0)
            compute(q0, kv0, ea0, idx0, contrib0, wv0, sidx0, ss0)
            idx_copy(2, idx0, is0).start()
            idx_copy(2, idx0, is0).wait()
            issue_gathers(idx0, q0, kv0, ea0, 2, gs0)
            wait_gathers(idx1, q1, kv1, ea1, 1, gs1)
            compute(q1, kv1, ea1, idx1, contrib1, wv1, sidx1, ss1)
            idx_copy(3, idx1, is1).start()

            def body2(t, carry):
                c0 = 2 * t
                c1 = c0 + 1
                idx_copy(c1, idx1, is1).wait()
                issue_gathers(idx1, q1, kv1, ea1, c1, gs1)
                wait_gathers(idx0, q0, kv0, ea0, c0, gs0)
                wait_scatters(contrib0, wv0, sidx0, ss0)
                compute(q0, kv0, ea0, idx0, contrib0, wv0, sidx0, ss0)
                idx_copy(c0 + 2, idx0, is0).start()
                idx_copy(c0 + 2, idx0, is0).wait()
                issue_gathers(idx0, q0, kv0, ea0, c0 + 2, gs0)
                wait_gathers(idx1, q1, kv1, ea1, c1, gs1)
                wait_scatters(contrib1, wv1, sidx1, ss1)
                compute(q1, kv1, ea1, idx1, contrib1, wv1, sidx1, ss1)
                idx_copy(c1 + 2, idx1, is1).start()
                return carry

            lax.fori_loop(1, (SROWS - 1) // 2, body2, 0)
            idx_copy(SROWS, idx1, is1).wait()
            clast = SROWS - 1
            wait_gathers(idx0, q0, kv0, ea0, clast, gs0)
            wait_scatters(contrib0, wv0, sidx0, ss0)
            compute(q0, kv0, ea0, idx0, contrib0, wv0, sidx0, ss0)
            wait_scatters(contrib0, wv0, sidx0, ss0)
            wait_scatters(contrib1, wv1, sidx1, ss1)

            plsc.subcore_barrier()
            oidx = et * H + head

            def wb(off, n, ns):
                pltpu.sync_copy(raw_acc.at[pl.ds(off, n)],
                                raw_out.at[oidx].at[pl.ds(off, n)])
                pltpu.sync_copy(s_acc.at[pl.ds(off, ns)],
                                s_out.at[oidx].at[pl.ds(off, ns)])
            dual(wb)
            plsc.subcore_barrier()


def _sc_edge(qw, qc, kvw, kvc, eaw, eac, ixw, ixc, z32, z1):
    mesh = plsc.VectorSubcoreMesh(core_axis_name="c", subcore_axis_name="s")
    fn = pl.kernel(
        _sc_edge_body,
        out_type=(
            jax.ShapeDtypeStruct((2 * H, NP, DH), jnp.float32),
            jax.ShapeDtypeStruct((2 * H, NPS), jnp.float32),
        ),
        mesh=mesh,
        compiler_params=pltpu.CompilerParams(
            needs_layout_passes=False, use_tc_tiling_on_sc=False),
        scratch_types=[
            pltpu.VMEM((2, ECH), jnp.int32),
            pltpu.VMEM((2, ECH), jnp.int32),
            pltpu.VMEM((ECH, DH), jnp.float32),
            pltpu.VMEM((ECH, DH), jnp.float32),
            pltpu.VMEM((ECH, 2 * DH), jnp.float32),
            pltpu.VMEM((ECH, 2 * DH), jnp.float32),
            pltpu.VMEM((ECH, DH), jnp.float32),
            pltpu.VMEM((ECH, DH), jnp.float32),
            pltpu.VMEM((ECH, DH), jnp.float32),
            pltpu.VMEM((ECH, DH), jnp.float32),
            pltpu.VMEM((ECH,), jnp.float32),
            pltpu.VMEM((ECH,), jnp.float32),
            pltpu.VMEM((ECH,), jnp.int32),
            pltpu.VMEM((ECH,), jnp.int32),
            pltpu.SemaphoreType.DMA,
            pltpu.SemaphoreType.DMA,
            pltpu.SemaphoreType.DMA,
            pltpu.SemaphoreType.DMA,
            pltpu.SemaphoreType.DMA,
            pltpu.SemaphoreType.DMA,
            pltpu.VMEM_SHARED((NP, DH), jnp.float32),
            pltpu.VMEM_SHARED((NPS,), jnp.float32),
        ],
    )
    return fn(qw, qc, kvw, kvc, eaw, eac, ixw, ixc, z32, z1)


def _post_body(raw_ref, s_ref, xp_ref, xa_ref, wa_ref, ba_ref, ombp_ref,
               gp_ref, bp_ref, abias_ref, omba_ref, ga_ref, bba_ref,
               hp_ref, ha_ref):
    f32 = jnp.float32
    o = None
    for h in range(H):
        rw = raw_ref[h]
        rc = raw_ref[H + h]
        sw = s_ref[:, h][:, None]
        sc_ = s_ref[:, H + h][:, None]
        agg = rw / (sw + 1e-16) + rc / (sc_ + 1e-16)
        g = jax.nn.gelu(agg)
        t = jnp.dot(g, wa_ref[h], preferred_element_type=f32)
        o = t if o is None else o + t
    res = o + ba_ref[...] + ombp_ref[...] * xp_ref[...]
    mu = jnp.mean(res, -1, keepdims=True)
    var = jnp.var(res, -1, keepdims=True)
    hp_ref[...] = gp_ref[...] * (res - mu) / jnp.sqrt(var + 1e-5) + bp_ref[...]

    ra = abias_ref[...] + omba_ref[...] * xa_ref[...]
    mua = jnp.mean(ra, -1, keepdims=True)
    vara = jnp.var(ra, -1, keepdims=True)
    ha_ref[...] = ga_ref[...] * (ra - mua) / jnp.sqrt(vara + 1e-5) + bba_ref[...]


def _post(raw, s, xp, xa, wa, ba, ombp, gp, bp, abias, omba, ga, bba):
    row = pl.BlockSpec((1, D), lambda nb: (0, 0))
    return pl.pallas_call(
        _post_body,
        grid=(NP // BLK,),
        in_specs=[
            pl.BlockSpec((2 * H, BLK, DH), lambda nb: (0, nb, 0)),
            pl.BlockSpec((BLK, 2 * H), lambda nb: (nb, 0)),
            pl.BlockSpec((BLK, D), lambda nb: (nb, 0)),
            pl.BlockSpec((BLK, D), lambda nb: (nb, 0)),
            pl.BlockSpec((H, DH, D), lambda nb: (0, 0, 0)),
            row, row, row, row, row, row, row, row,
        ],
        out_specs=[
            pl.BlockSpec((BLK, D), lambda nb: (nb, 0)),
            pl.BlockSpec((BLK, D), lambda nb: (nb, 0)),
        ],
        out_shape=[
            jax.ShapeDtypeStruct((NP, D), jnp.float32),
            jax.ShapeDtypeStruct((NA, D), jnp.float32),
        ],
    )(raw, s, xp, xa, wa, ba, ombp, gp, bp, abias, omba, ga, bba)


def kernel(x_paper, x_author, edge_index_writes, edge_index_cites,
           edge_t2v_writes, edge_t2v_cites, params):
    f32 = jnp.float32
    inv = 1.0 / math.sqrt(float(DH))

    def prep_ea(e, lin):
        e_t = jnp.pad(e.astype(f32), ((0, 0), (0, 16 - EDIM))).T
        w = jnp.pad(lin["w"].astype(f32), ((0, 16 - EDIM), (0, 0)))
        ea = _ea_proj(e_t, w, lin["b"].reshape(1, D))
        return ea.reshape(EW, H, DH).transpose(1, 0, 2)

    eaw = prep_ea(edge_t2v_writes, params["edge_lin"]["writes"])
    eac = prep_ea(edge_t2v_cites, params["edge_lin"]["cites"])

    def prep_ix(ei):
        s_ = ei[0].astype(jnp.int32).reshape(NSUB, SROWS, ECH)
        d_ = ei[1].astype(jnp.int32).reshape(NSUB, SROWS, ECH)
        return jnp.stack([d_, s_], axis=2)

    ixw = prep_ix(edge_index_writes)
    ixc = prep_ix(edge_index_cites)

    z32 = jnp.zeros((NP, DH), f32)
    z1 = jnp.zeros((NPS,), f32)

    def per_head(w):
        return w.reshape(D, H, DH).transpose(1, 0, 2)

    def kv_weights(wk, bk, wv, bv, ar, mr):
        wk_eff = jnp.einsum("hkd,hdf->hkf", per_head(wk), ar)
        bk_eff = jnp.einsum("hd,hdf->hf", bk.reshape(H, DH), ar)
        wv_eff = jnp.einsum("hkd,hdf->hkf", per_head(wv), mr)
        bv_eff = jnp.einsum("hd,hdf->hf", bv.reshape(H, DH), mr)
        w = jnp.concatenate([wk_eff, wv_eff], axis=2)
        w = w.transpose(1, 0, 2).reshape(D, 2 * D)
        b = jnp.concatenate([bk_eff, bv_eff], axis=1).reshape(1, 2 * D)
        return w, b

    def headify(t, width):
        return t.reshape(-1, H, width).transpose(1, 0, 2)

    h_p = x_paper
    h_a = x_author
    for lp in params["layers"]:
        sv_w = jnp.repeat(lp["p_rel"]["writes"] * inv, DH)
        sv_c = jnp.repeat(lp["p_rel"]["cites"] * inv, DH)
        wq = lp["q"]["paper"]["w"]
        bq = lp["q"]["paper"]["b"]
        wkv_c, bkv_c = kv_weights(
            lp["k"]["paper"]["w"], lp["k"]["paper"]["b"],
            lp["v"]["paper"]["w"], lp["v"]["paper"]["b"],
            lp["a_rel"]["cites"], lp["m_rel"]["cites"])
        w_paper = jnp.concatenate(
            [wq * sv_w[None, :], wq * sv_c[None, :], wkv_c], axis=1)
        b_paper = jnp.concatenate(
            [(bq * sv_w).reshape(1, D), (bq * sv_c).reshape(1, D), bkv_c],
            axis=1)
        qw0, qc0, kvc0 = _paper_proj(h_p, w_paper, b_paper)
        qw = headify(qw0, DH)
        qc = headify(qc0, DH)
        kvc = headify(kvc0, 2 * DH)

        wkv_w, bkv_w = kv_weights(
            lp["k"]["author"]["w"], lp["k"]["author"]["b"],
            lp["v"]["author"]["w"], lp["v"]["author"]["b"],
            lp["a_rel"]["writes"], lp["m_rel"]["writes"])
        kvw = headify(_author_proj(h_a, wkv_w, bkv_w), 2 * DH)

        raw, s = _sc_edge(qw, qc, kvw, kvc, eaw, eac, ixw, ixc, z32, z1)

        beta_p = jax.nn.sigmoid(lp["skip"]["paper"])
        beta_a = jax.nn.sigmoid(lp["skip"]["author"])
        wa = (lp["a"]["paper"]["w"].reshape(H, DH, D)) * beta_p
        ba = (lp["a"]["paper"]["b"] * beta_p).reshape(1, D)
        ombp = jnp.full((1, D), 1.0 - beta_p, f32)
        gp = params["norm"]["paper"]["g"].reshape(1, D)
        bp = params["norm"]["paper"]["b"].reshape(1, D)
        abias = (beta_a * lp["a"]["author"]["b"]).reshape(1, D)
        omba = jnp.full((1, D), 1.0 - beta_a, f32)
        ga = params["norm"]["author"]["g"].reshape(1, D)
        bba = params["norm"]["author"]["b"].reshape(1, D)

        h_p, h_a = _post(raw, s.transpose(1, 0)[:NP], h_p, h_a,
                         wa, ba, ombp, gp, bp, abias, omba, ga, bba)

    return (h_p, h_a)
